# baseline (speedup 1.0000x reference)
import functools

import jax
import jax.numpy as jnp
from jax import lax
from jax.experimental import pallas as pl
from jax.experimental.pallas import tpu as pltpu
from jax.experimental.pallas import tpu_sc as plsc

_D = 128
_S = 4096
_T = 50
_NW = 32
_SPW = _S // _NW
_NSLOT = _T
_NBUF = 7
_LOOK = 4


def _emb_body(idx_hbm, table_hbm, out_hbm, idx_v, rows_v, *sems):
    gs = sems[:_NBUF]
    ss = sems[_NBUF:]
    wid = lax.axis_index("s") * 2 + lax.axis_index("c")
    sbase = wid * _SPW

    pltpu.sync_copy(idx_hbm.at[wid], idx_v)

    def gather(c, b):
        pltpu.async_copy(table_hbm.at[idx_v.at[c]], rows_v.at[b], gs[b])

    def gwait(c, b):
        pltpu.make_async_copy(table_hbm.at[idx_v.at[c]], rows_v.at[b], gs[b]).wait()

    def store(c, b):
        pltpu.async_copy(
            rows_v.at[b], out_hbm.at[c].at[pl.ds(sbase, _SPW)], ss[b]
        )

    def swait(c, b):
        pltpu.make_async_copy(
            rows_v.at[b], out_hbm.at[c].at[pl.ds(sbase, _SPW)], ss[b]
        ).wait()

    def slot(c, b, do_swait, do_gather):
        gwait(c, b)
        store(c, b)
        if do_swait:
            swait(c - (_NBUF - _LOOK), (b + _LOOK) % _NBUF)
        if do_gather:
            gather(c + _LOOK, (b + _LOOK) % _NBUF)

    for c in range(_LOOK):
        gather(c, c)
    for c in range(_NBUF - _LOOK):
        slot(c, c, False, True)

    first = _NBUF - _LOOK
    last = _NSLOT - _LOOK
    nloop = (last - first) // _NBUF

    def body(t, carry):
        c0 = first + t * _NBUF
        for i in range(_NBUF):
            slot(c0 + i, (first + i) % _NBUF, True, True)
        return carry

    lax.fori_loop(0, nloop, body, 0)

    for c in range(first + nloop * _NBUF, last):
        slot(c, c % _NBUF, True, True)

    for c in range(last, _NSLOT):
        slot(c, c % _NBUF, True, False)
    for c in range(_NSLOT - (_NBUF - _LOOK), _NSLOT):
        swait(c, c % _NBUF)


_emb = functools.partial(
    pl.kernel,
    mesh=plsc.VectorSubcoreMesh(core_axis_name="c", subcore_axis_name="s"),
    out_type=jax.ShapeDtypeStruct((_T, _S, _D), jnp.float32),
    scratch_types=[
        pltpu.VMEM((_T, _SPW), jnp.int32),
        pltpu.VMEM((_NBUF, _SPW, _D), jnp.float32),
    ] + [pltpu.SemaphoreType.DMA] * (2 * _NBUF),
)(_emb_body)


@jax.jit
def kernel(token_ids, weight):
    idx_w = (
        token_ids.astype(jnp.int32)
        .reshape(_NW, _SPW, _T)
        .transpose(0, 2, 1)
    )
    out = _emb(idx_w, weight)
    return out.transpose(1, 0, 2)


# speedup vs baseline: 1.0089x; 1.0089x over previous
import functools

import jax
import jax.numpy as jnp
from jax import lax
from jax.experimental import pallas as pl
from jax.experimental.pallas import tpu as pltpu
from jax.experimental.pallas import tpu_sc as plsc

_D = 128
_S = 4096
_T = 50
_NW = 32
_SPW = _S // _NW
_NSLOT = _T
_NBUF = 7
_LOOK = 6


def _emb_body(idx_hbm, table_hbm, out_hbm, idx_v, rows_v, *sems):
    gs = sems[:_NBUF]
    ss = sems[_NBUF:]
    wid = lax.axis_index("s") * 2 + lax.axis_index("c")
    sbase = wid * _SPW

    pltpu.sync_copy(idx_hbm.at[wid], idx_v)

    def gather(c, b):
        pltpu.async_copy(table_hbm.at[idx_v.at[c]], rows_v.at[b], gs[b])

    def gwait(c, b):
        pltpu.make_async_copy(table_hbm.at[idx_v.at[c]], rows_v.at[b], gs[b]).wait()

    def store(c, b):
        pltpu.async_copy(
            rows_v.at[b], out_hbm.at[c].at[pl.ds(sbase, _SPW)], ss[b]
        )

    def swait(c, b):
        pltpu.make_async_copy(
            rows_v.at[b], out_hbm.at[c].at[pl.ds(sbase, _SPW)], ss[b]
        ).wait()

    def slot(c, b, do_swait, do_gather):
        gwait(c, b)
        store(c, b)
        if do_swait:
            swait(c - (_NBUF - _LOOK), (b + _LOOK) % _NBUF)
        if do_gather:
            gather(c + _LOOK, (b + _LOOK) % _NBUF)

    for c in range(_LOOK):
        gather(c, c)
    for c in range(_NBUF - _LOOK):
        slot(c, c, False, True)

    first = _NBUF - _LOOK
    last = _NSLOT - _LOOK
    nloop = (last - first) // _NBUF

    def body(t, carry):
        c0 = first + t * _NBUF
        for i in range(_NBUF):
            slot(c0 + i, (first + i) % _NBUF, True, True)
        return carry

    lax.fori_loop(0, nloop, body, 0)

    for c in range(first + nloop * _NBUF, last):
        slot(c, c % _NBUF, True, True)

    for c in range(last, _NSLOT):
        slot(c, c % _NBUF, True, False)
    for c in range(_NSLOT - (_NBUF - _LOOK), _NSLOT):
        swait(c, c % _NBUF)


_emb = functools.partial(
    pl.kernel,
    mesh=plsc.VectorSubcoreMesh(core_axis_name="c", subcore_axis_name="s"),
    out_type=jax.ShapeDtypeStruct((_T, _S, _D), jnp.float32),
    scratch_types=[
        pltpu.VMEM((_T, _SPW), jnp.int32),
        pltpu.VMEM((_NBUF, _SPW, _D), jnp.float32),
    ] + [pltpu.SemaphoreType.DMA] * (2 * _NBUF),
)(_emb_body)


@jax.jit
def kernel(token_ids, weight):
    idx_w = (
        token_ids.astype(jnp.int32)
        .reshape(_NW, _SPW, _T)
        .transpose(0, 2, 1)
    )
    out = _emb(idx_w, weight)
    return out.transpose(1, 0, 2)
